# baseline (speedup 1.0000x reference)
import dataclasses
import functools

import jax
import jax.numpy as jnp
import numpy as np
from jax import lax
from jax.experimental import pallas as pl
from jax.experimental.pallas import tpu as pltpu
from jax.experimental.pallas import tpu_sc as plsc

N = 10000
E = 320000
HID = 128
H = 8
OUT = 16
NL = 3
FFN = 128

NC = 2
NS = 16
NW = NC * NS
CH = 128
NCHUNK = E // CH
MAXC = -(-NCHUNK // NW)
NP = N + 640
BZ = 80
NBZ = NP // BZ

_f32 = jnp.float32


def _sds(shape):
    return jax.ShapeDtypeStruct(shape, _f32)


def _sc_params():
    cp = pltpu.CompilerParams()
    if "needs_layout_passes" in pltpu.CompilerParams.__dataclass_fields__:
        cp = dataclasses.replace(cp, needs_layout_passes=False)
    return cp


def _sc_gather(xs, xd, src, dst):
    mesh = plsc.VectorSubcoreMesh(core_axis_name="c", subcore_axis_name="s")

    @functools.partial(
        pl.kernel,
        out_type=(_sds((E, HID)), _sds((E, HID))),
        mesh=mesh,
        scratch_types=[
            pltpu.VMEM((CH,), jnp.int32),
            pltpu.VMEM((CH,), jnp.int32),
            pltpu.VMEM((CH, HID), _f32),
            pltpu.VMEM((CH, HID), _f32),
            pltpu.SemaphoreType.DMA,
            pltpu.SemaphoreType.DMA,
        ],
    )
    def k(xs_hbm, xd_hbm, src_hbm, dst_hbm, os_hbm, od_hbm,
          si_v, di_v, rs_v, rd_v, sem_a, sem_b):
        cid = lax.axis_index("c")
        sid = lax.axis_index("s")
        wid = sid * NC + cid
        per = NCHUNK // NW
        rem = NCHUNK % NW
        start = wid * per + jnp.minimum(wid, rem)
        cnt = per + jnp.where(wid < rem, 1, 0)

        @pl.loop(0, MAXC)
        def _(i):
            @pl.when(i < cnt)
            def _():
                base = (start + i) * CH
                pltpu.sync_copy(src_hbm.at[pl.ds(base, CH)], si_v)
                pltpu.sync_copy(dst_hbm.at[pl.ds(base, CH)], di_v)
                a = pltpu.async_copy(xs_hbm.at[si_v], rs_v, sem_a)
                b = pltpu.async_copy(xd_hbm.at[di_v], rd_v, sem_b)
                a.wait()
                b.wait()
                pltpu.sync_copy(rs_v, os_hbm.at[pl.ds(base, CH)])
                pltpu.sync_copy(rd_v, od_hbm.at[pl.ds(base, CH)])

    return k(xs, xd, src, dst)


def _sc_scatter(num_c, ex_f, dst, zt):
    mesh = plsc.VectorSubcoreMesh(core_axis_name="c", subcore_axis_name="s")

    @functools.partial(
        pl.kernel,
        out_type=_sds((NC * NP, HID)),
        mesh=mesh,
        scratch_types=[
            pltpu.VMEM((CH,), jnp.int32),
            pltpu.VMEM((CH,), jnp.int32),
            pltpu.VMEM((CH, HID), _f32),
            pltpu.VMEM((CH, HID), _f32),
            pltpu.VMEM((CH * H,), _f32),
            pltpu.VMEM_SHARED((NP, HID), _f32),
            pltpu.SemaphoreType.DMA,
            pltpu.SemaphoreType.DMA,
        ],
        compiler_params=_sc_params(),
    )
    def k(nc_hbm, ex_hbm, dst_hbm, zt_hbm, ot_hbm,
          idx_v, didx_v, dat_v, den_v, ex_v, tab_s, sem_a, sem_b):
        cid = lax.axis_index("c")
        sid = lax.axis_index("s")
        wid = sid * NC + cid
        per = NCHUNK // NW
        rem = NCHUNK % NW
        start = wid * per + jnp.minimum(wid, rem)
        cnt = per + jnp.where(wid < rem, 1, 0)
        zv16 = jnp.zeros((16,), _f32)
        lane = lax.iota(jnp.int32, 16)

        @pl.loop(0, CH)
        def _(r):
            @pl.loop(0, HID // 16)
            def _(c):
                den_v[r, pl.ds(c * 16, 16)] = zv16

        @pl.loop(0, -(-NBZ // NS))
        def _(j):
            blk = j * NS + sid

            @pl.when(blk < NBZ)
            def _():
                pltpu.sync_copy(zt_hbm.at[pl.ds(blk * BZ, BZ)],
                                dat_v.at[pl.ds(0, BZ)])
                pltpu.sync_copy(dat_v.at[pl.ds(0, BZ)],
                                tab_s.at[pl.ds(blk * BZ, BZ)])

        plsc.subcore_barrier()

        @pl.loop(0, MAXC)
        def _(i):
            @pl.when(i < cnt)
            def _():
                base = (start + i) * CH
                pltpu.sync_copy(dst_hbm.at[pl.ds(base, CH)], idx_v)
                pltpu.sync_copy(nc_hbm.at[pl.ds(base, CH)], dat_v)
                pltpu.sync_copy(ex_hbm.at[pl.ds(base * H, CH * H)], ex_v)
                add_n = pltpu.async_copy(dat_v, tab_s.at[idx_v], sem_a,
                                         add=True)

                @pl.loop(0, CH // 16)
                def _(g):
                    row = lane + g * 16
                    dvec = idx_v[pl.ds(g * 16, 16)]
                    didx_v[pl.ds(g * 16, 16)] = N + dvec // 16
                    colb = lax.rem(dvec, 16) * H
                    for h in range(H):
                        vals = plsc.load_gather(ex_v, [row * H + h])
                        plsc.store_scatter(den_v, [row, colb + h], vals)

                add_d = pltpu.async_copy(den_v, tab_s.at[didx_v], sem_b,
                                         add=True)
                add_n.wait()
                add_d.wait()

                @pl.loop(0, CH // 16)
                def _(g):
                    row = lane + g * 16
                    dvec = idx_v[pl.ds(g * 16, 16)]
                    colb = lax.rem(dvec, 16) * H
                    for h in range(H):
                        plsc.store_scatter(den_v, [row, colb + h], zv16)

        plsc.subcore_barrier()

        @pl.loop(0, -(-NBZ // NS))
        def _(j):
            blk = j * NS + sid

            @pl.when(blk < NBZ)
            def _():
                pltpu.sync_copy(tab_s.at[pl.ds(blk * BZ, BZ)],
                                dat_v.at[pl.ds(0, BZ)])
                pltpu.sync_copy(dat_v.at[pl.ds(0, BZ)],
                                ot_hbm.at[pl.ds(cid * NP + blk * BZ, BZ)])

    return k(num_c, ex_f, dst, zt)


def _tc_project(x, wl, wr):
    def body(x_ref, wl_ref, wr_ref, xs_ref, xd_ref):
        xv = x_ref[...]
        xs_ref[...] = jnp.dot(xv, wl_ref[...], preferred_element_type=_f32)
        xd_ref[...] = jnp.dot(xv, wr_ref[...], preferred_element_type=_f32)

    return pl.pallas_call(
        body, out_shape=(_sds((N, HID)), _sds((N, HID))),
    )(x, wl, wr)


_EB = 2000


def _tc_edge_a(e_in, xs_src, xd_dst, wep, bep, we, attb, sel, project):
    nblk = E // _EB
    in_ch = e_in.shape[1]

    def body(e_ref, xs_ref, xd_ref, wep_ref, bep_ref, we_ref, attb_ref,
             sel_ref, m_ref, log_ref, gmax_ref):
        if project:
            e = jnp.dot(e_ref[...], wep_ref[...],
                        preferred_element_type=_f32) + bep_ref[...]
        else:
            e = e_ref[...]
        ee = jnp.dot(e, we_ref[...], preferred_element_type=_f32)
        m = xs_ref[...] + xd_ref[...] + ee
        g = jnp.where(m >= 0, m, 0.2 * m)
        logits = jnp.dot(g * attb_ref[...], sel_ref[...],
                         preferred_element_type=_f32)
        m_ref[...] = m
        log_ref[...] = logits
        bm = jnp.max(logits, axis=0, keepdims=True)
        i = pl.program_id(0)

        @pl.when(i == 0)
        def _():
            gmax_ref[...] = bm

        @pl.when(i != 0)
        def _():
            gmax_ref[...] = jnp.maximum(gmax_ref[...], bm)

    wspec = lambda s: pl.BlockSpec(s, lambda i: (0, 0))
    return pl.pallas_call(
        body,
        grid=(nblk,),
        in_specs=[
            pl.BlockSpec((_EB, in_ch), lambda i: (i, 0)),
            pl.BlockSpec((_EB, HID), lambda i: (i, 0)),
            pl.BlockSpec((_EB, HID), lambda i: (i, 0)),
            wspec((16, HID)), wspec((1, HID)), wspec((HID, HID)),
            wspec((1, HID)), wspec((HID, H)),
        ],
        out_specs=[
            pl.BlockSpec((_EB, HID), lambda i: (i, 0)),
            pl.BlockSpec((_EB, H), lambda i: (i, 0)),
            pl.BlockSpec((1, H), lambda i: (0, 0)),
        ],
        out_shape=(_sds((E, HID)), _sds((E, H)), _sds((1, H))),
    )(e_in, xs_src, xd_dst, wep, bep, we, attb, sel)


def _tc_edge_b(e_in, m, wep, bep, weu, beu, project):
    nblk = E // _EB
    in_ch = e_in.shape[1]

    def body(e_ref, m_ref, wep_ref, bep_ref, weu_ref, beu_ref, enew_ref):
        if project:
            e = jnp.dot(e_ref[...], wep_ref[...],
                        preferred_element_type=_f32) + bep_ref[...]
        else:
            e = e_ref[...]
        r = jnp.maximum(m_ref[...], 0.0)
        enew_ref[...] = e + jnp.dot(r, weu_ref[...],
                                    preferred_element_type=_f32) + beu_ref[...]

    wspec = lambda s: pl.BlockSpec(s, lambda i: (0, 0))
    return pl.pallas_call(
        body,
        grid=(nblk,),
        in_specs=[
            pl.BlockSpec((_EB, in_ch), lambda i: (i, 0)),
            pl.BlockSpec((_EB, HID), lambda i: (i, 0)),
            wspec((16, HID)), wspec((1, HID)), wspec((HID, HID)),
            wspec((1, HID)),
        ],
        out_specs=pl.BlockSpec((_EB, HID), lambda i: (i, 0)),
        out_shape=_sds((E, HID)),
    )(e_in, m, wep, bep, weu, beu)


def _tc_contrib(logits, gmax, xs_src, selt):
    nblk = E // _EB

    def body(log_ref, xs_ref, gmax_ref, selt_ref, num_ref, ex_ref):
        ex = jnp.exp(log_ref[...] - gmax_ref[...])
        exb = jnp.dot(ex, selt_ref[...], preferred_element_type=_f32)
        num_ref[...] = exb * xs_ref[...]
        ex_ref[...] = ex

    return pl.pallas_call(
        body,
        grid=(nblk,),
        in_specs=[
            pl.BlockSpec((_EB, H), lambda i: (i, 0)),
            pl.BlockSpec((_EB, HID), lambda i: (i, 0)),
            pl.BlockSpec((1, H), lambda i: (0, 0)),
            pl.BlockSpec((H, HID), lambda i: (0, 0)),
        ],
        out_specs=[
            pl.BlockSpec((_EB, HID), lambda i: (i, 0)),
            pl.BlockSpec((_EB, H), lambda i: (i, 0)),
        ],
        out_shape=(_sds((E, HID)), _sds((E, H))),
    )(logits, xs_src, gmax, selt)


def _tc_node(tab, den8, x, wres, bres, w1, b1, w2, b2, selt, km, ln):
    def body(*refs):
        if ln is not None:
            (tab_ref, den_ref, x_ref, wres_ref, bres_ref, w1_ref, b1_ref,
             w2_ref, b2_ref, selt_ref, km_ref, g_ref, be_ref, o_ref) = refs
        else:
            (tab_ref, den_ref, x_ref, wres_ref, bres_ref, w1_ref, b1_ref,
             w2_ref, b2_ref, selt_ref, km_ref, o_ref) = refs
        num = tab_ref[0:N, :] + tab_ref[NP:NP + N, :]
        den = den_ref[0:N, :] + den_ref[N:2 * N, :]
        invb = jnp.dot(1.0 / (den + 1e-16), selt_ref[...],
                       preferred_element_type=_f32)
        agg = jnp.dot(num * invb, km_ref[...], preferred_element_type=_f32)
        out = agg + jnp.dot(x_ref[...], wres_ref[...],
                            preferred_element_type=_f32) + bres_ref[...]
        h1 = jnp.maximum(jnp.dot(out, w1_ref[...],
                                 preferred_element_type=_f32) + b1_ref[...], 0.0)
        out = out + jnp.dot(h1, w2_ref[...],
                            preferred_element_type=_f32) + b2_ref[...]
        if ln is not None:
            mu = jnp.mean(out, axis=1, keepdims=True)
            v = jnp.mean((out - mu) ** 2, axis=1, keepdims=True)
            out = (out - mu) / jnp.sqrt(v + 1e-5) * g_ref[...] + be_ref[...]
        o_ref[...] = out

    args = [tab, den8, x, wres, bres, w1, b1, w2, b2, selt, km]
    if ln is not None:
        args += [ln[0], ln[1]]
    return pl.pallas_call(body, out_shape=_sds((N, OUT)))(*args)


def kernel(mesh_enc, edge_index, edge_attr, params):
    src = edge_index[0]
    dst = edge_index[1]
    zt = jnp.zeros((NP, HID), _f32)

    sel_np = np.kron(np.eye(H, dtype=np.float32),
                     np.ones((OUT, 1), dtype=np.float32))
    sel = jnp.asarray(sel_np)
    selt = jnp.asarray(sel_np.T)
    sel16 = jnp.asarray(np.concatenate(
        [np.eye(H, dtype=np.float32),
         np.zeros((H, OUT - H), dtype=np.float32)], axis=1))
    km = jnp.asarray(np.kron(np.ones((H, 1), dtype=np.float32) / H,
                             np.eye(OUT, dtype=np.float32)))

    wep = params["W_ep"]
    bep = params["b_ep"].reshape(1, HID)

    x = mesh_enc
    e = edge_attr
    for l in range(NL):
        p = params["layers"][l]
        attb = p["att"].reshape(1, H * OUT)
        xs, xd = _tc_project(x, p["Wl"], p["Wr"])
        xs_src, xd_dst = _sc_gather(xs, xd, src, dst)
        m, logits, gmax = _tc_edge_a(
            e, xs_src, xd_dst, wep, bep, p["We"], attb, sel,
            project=(l == 0))
        num_c, ex = _tc_contrib(logits, gmax, xs_src, selt)
        enew = _tc_edge_b(e, m, wep, bep, p["Weu"],
                          p["beu"].reshape(1, HID), project=(l == 0))
        tab = _sc_scatter(num_c, ex.reshape(E * H), dst, zt)
        den8 = jnp.concatenate(
            [tab[N:N + N // 16].reshape(N, H),
             tab[NP + N:NP + N + N // 16].reshape(N, H)], axis=0)
        ln = None
        if l < NL - 1:
            ln = (params["norms"][l]["g"].reshape(1, OUT),
                  params["norms"][l]["b"].reshape(1, OUT))
        x = _tc_node(tab, den8, x, p["Wres"], p["bres"].reshape(1, OUT),
                     p["W1"], p["b1"].reshape(1, FFN),
                     p["W2"], p["b2"].reshape(1, OUT), selt, km, ln)
        e = enew
    return x


# speedup vs baseline: 1.0087x; 1.0087x over previous
import dataclasses
import functools

import jax
import jax.numpy as jnp
import numpy as np
from jax import lax
from jax.experimental import pallas as pl
from jax.experimental.pallas import tpu as pltpu
from jax.experimental.pallas import tpu_sc as plsc

N = 10000
E = 320000
HID = 128
H = 8
OUT = 16
NL = 3
FFN = 128

NC = 2
NS = 16
NW = NC * NS
CH = 128
NCHUNK = E // CH
MAXC = -(-NCHUNK // NW)
NP = N + 640
BZ = 80
NBZ = NP // BZ

_f32 = jnp.float32


def _sds(shape):
    return jax.ShapeDtypeStruct(shape, _f32)


def _sc_params():
    cp = pltpu.CompilerParams()
    if "needs_layout_passes" in pltpu.CompilerParams.__dataclass_fields__:
        cp = dataclasses.replace(cp, needs_layout_passes=False)
    return cp


def _sc_gather(xs, xd, src, dst):
    mesh = plsc.VectorSubcoreMesh(core_axis_name="c", subcore_axis_name="s")

    @functools.partial(
        pl.kernel,
        out_type=(_sds((E, HID)), _sds((E, HID))),
        mesh=mesh,
        scratch_types=[
            pltpu.VMEM((CH,), jnp.int32),
            pltpu.VMEM((CH,), jnp.int32),
            pltpu.VMEM((CH, HID), _f32),
            pltpu.VMEM((CH, HID), _f32),
            pltpu.SemaphoreType.DMA,
            pltpu.SemaphoreType.DMA,
        ],
    )
    def k(xs_hbm, xd_hbm, src_hbm, dst_hbm, os_hbm, od_hbm,
          si_v, di_v, rs_v, rd_v, sem_a, sem_b):
        cid = lax.axis_index("c")
        sid = lax.axis_index("s")
        wid = sid * NC + cid
        per = NCHUNK // NW
        rem = NCHUNK % NW
        start = wid * per + jnp.minimum(wid, rem)
        cnt = per + jnp.where(wid < rem, 1, 0)

        @pl.loop(0, MAXC)
        def _(i):
            @pl.when(i < cnt)
            def _():
                base = (start + i) * CH
                pltpu.sync_copy(src_hbm.at[pl.ds(base, CH)], si_v)
                pltpu.sync_copy(dst_hbm.at[pl.ds(base, CH)], di_v)
                a = pltpu.async_copy(xs_hbm.at[si_v], rs_v, sem_a)
                b = pltpu.async_copy(xd_hbm.at[di_v], rd_v, sem_b)
                a.wait()
                b.wait()
                pltpu.sync_copy(rs_v, os_hbm.at[pl.ds(base, CH)])
                pltpu.sync_copy(rd_v, od_hbm.at[pl.ds(base, CH)])

    return k(xs, xd, src, dst)


def _sc_scatter(num_c, ex_f, dst, zt):
    mesh = plsc.VectorSubcoreMesh(core_axis_name="c", subcore_axis_name="s")

    @functools.partial(
        pl.kernel,
        out_type=_sds((NC * NP, HID)),
        mesh=mesh,
        scratch_types=[
            pltpu.VMEM((CH,), jnp.int32),
            pltpu.VMEM((CH,), jnp.int32),
            pltpu.VMEM((CH, HID), _f32),
            pltpu.VMEM((CH, HID), _f32),
            pltpu.VMEM((CH * H,), _f32),
            pltpu.VMEM_SHARED((NP, HID), _f32),
            pltpu.SemaphoreType.DMA,
            pltpu.SemaphoreType.DMA,
        ],
        compiler_params=_sc_params(),
    )
    def k(nc_hbm, ex_hbm, dst_hbm, zt_hbm, ot_hbm,
          idx_v, didx_v, dat_v, den_v, ex_v, tab_s, sem_a, sem_b):
        cid = lax.axis_index("c")
        sid = lax.axis_index("s")
        wid = sid * NC + cid
        per = NCHUNK // NW
        rem = NCHUNK % NW
        start = wid * per + jnp.minimum(wid, rem)
        cnt = per + jnp.where(wid < rem, 1, 0)
        zv16 = jnp.zeros((16,), _f32)
        lane = lax.iota(jnp.int32, 16)

        @pl.loop(0, CH)
        def _(r):
            @pl.loop(0, HID // 16)
            def _(c):
                den_v[r, pl.ds(c * 16, 16)] = zv16

        @pl.loop(0, -(-NBZ // NS))
        def _(j):
            blk = j * NS + sid

            @pl.when(blk < NBZ)
            def _():
                pltpu.sync_copy(zt_hbm.at[pl.ds(blk * BZ, BZ)],
                                dat_v.at[pl.ds(0, BZ)])
                pltpu.sync_copy(dat_v.at[pl.ds(0, BZ)],
                                tab_s.at[pl.ds(blk * BZ, BZ)])

        plsc.subcore_barrier()

        @pl.loop(0, MAXC)
        def _(i):
            @pl.when(i < cnt)
            def _():
                base = (start + i) * CH
                pltpu.sync_copy(dst_hbm.at[pl.ds(base, CH)], idx_v)
                pltpu.sync_copy(nc_hbm.at[pl.ds(base, CH)], dat_v)
                pltpu.sync_copy(ex_hbm.at[pl.ds(base * H, CH * H)], ex_v)
                add_n = pltpu.async_copy(dat_v, tab_s.at[idx_v], sem_a,
                                         add=True)

                @pl.loop(0, CH // 16)
                def _(g):
                    row = lane + g * 16
                    dvec = idx_v[pl.ds(g * 16, 16)]
                    didx_v[pl.ds(g * 16, 16)] = N + dvec // 16
                    colb = lax.rem(dvec, 16) * H
                    for h in range(H):
                        vals = plsc.load_gather(ex_v, [row * H + h])
                        plsc.store_scatter(den_v, [row, colb + h], vals)

                add_d = pltpu.async_copy(den_v, tab_s.at[didx_v], sem_b,
                                         add=True)
                add_n.wait()
                add_d.wait()

                @pl.loop(0, CH // 16)
                def _(g):
                    row = lane + g * 16
                    dvec = idx_v[pl.ds(g * 16, 16)]
                    colb = lax.rem(dvec, 16) * H
                    for h in range(H):
                        plsc.store_scatter(den_v, [row, colb + h], zv16)

        plsc.subcore_barrier()

        @pl.loop(0, -(-NBZ // NS))
        def _(j):
            blk = j * NS + sid

            @pl.when(blk < NBZ)
            def _():
                pltpu.sync_copy(tab_s.at[pl.ds(blk * BZ, BZ)],
                                dat_v.at[pl.ds(0, BZ)])
                pltpu.sync_copy(dat_v.at[pl.ds(0, BZ)],
                                ot_hbm.at[pl.ds(cid * NP + blk * BZ, BZ)])

    return k(num_c, ex_f, dst, zt)


def _tc_project(x, wl, wr):
    def body(x_ref, wl_ref, wr_ref, xs_ref, xd_ref):
        xv = x_ref[...]
        xs_ref[...] = jnp.dot(xv, wl_ref[...], preferred_element_type=_f32)
        xd_ref[...] = jnp.dot(xv, wr_ref[...], preferred_element_type=_f32)

    return pl.pallas_call(
        body, out_shape=(_sds((N, HID)), _sds((N, HID))),
    )(x, wl, wr)


_EB = 2000


def _tc_edge(e_in, xs_src, xd_dst, wep, bep, we, attb, sel, weu, beu, project):
    nblk = E // _EB
    in_ch = e_in.shape[1]

    def body(e_ref, xs_ref, xd_ref, wep_ref, bep_ref, we_ref, attb_ref,
             sel_ref, weu_ref, beu_ref, enew_ref, log_ref, gmax_ref):
        if project:
            e = jnp.dot(e_ref[...], wep_ref[...],
                        preferred_element_type=_f32) + bep_ref[...]
        else:
            e = e_ref[...]
        bf = jnp.bfloat16
        ee = jnp.dot(e.astype(bf), we_ref[...].astype(bf),
                     preferred_element_type=_f32)
        m = xs_ref[...] + xd_ref[...] + ee
        g = jnp.where(m >= 0, m, 0.2 * m)
        logits = jnp.dot(g * attb_ref[...], sel_ref[...],
                         preferred_element_type=_f32)
        r = jnp.maximum(g, 0.0)
        enew_ref[...] = e + jnp.dot(r.astype(bf), weu_ref[...].astype(bf),
                                    preferred_element_type=_f32) + beu_ref[...]
        log_ref[...] = logits
        bm = jnp.max(logits, axis=0, keepdims=True)
        i = pl.program_id(0)

        @pl.when(i == 0)
        def _():
            gmax_ref[...] = bm

        @pl.when(i != 0)
        def _():
            gmax_ref[...] = jnp.maximum(gmax_ref[...], bm)

    wspec = lambda s: pl.BlockSpec(s, lambda i: (0, 0))
    return pl.pallas_call(
        body,
        grid=(nblk,),
        in_specs=[
            pl.BlockSpec((_EB, in_ch), lambda i: (i, 0)),
            pl.BlockSpec((_EB, HID), lambda i: (i, 0)),
            pl.BlockSpec((_EB, HID), lambda i: (i, 0)),
            wspec((16, HID)), wspec((1, HID)), wspec((HID, HID)),
            wspec((1, HID)), wspec((HID, H)), wspec((HID, HID)),
            wspec((1, HID)),
        ],
        out_specs=[
            pl.BlockSpec((_EB, HID), lambda i: (i, 0)),
            pl.BlockSpec((_EB, H), lambda i: (i, 0)),
            pl.BlockSpec((1, H), lambda i: (0, 0)),
        ],
        out_shape=(_sds((E, HID)), _sds((E, H)), _sds((1, H))),
    )(e_in, xs_src, xd_dst, wep, bep, we, attb, sel, weu, beu)


def _tc_contrib(logits, gmax, xs_src, selt):
    nblk = E // _EB

    def body(log_ref, xs_ref, gmax_ref, selt_ref, num_ref, ex_ref):
        ex = jnp.exp(log_ref[...] - gmax_ref[...])
        exb = jnp.dot(ex, selt_ref[...], preferred_element_type=_f32)
        num_ref[...] = exb * xs_ref[...]
        ex_ref[...] = ex

    return pl.pallas_call(
        body,
        grid=(nblk,),
        in_specs=[
            pl.BlockSpec((_EB, H), lambda i: (i, 0)),
            pl.BlockSpec((_EB, HID), lambda i: (i, 0)),
            pl.BlockSpec((1, H), lambda i: (0, 0)),
            pl.BlockSpec((H, HID), lambda i: (0, 0)),
        ],
        out_specs=[
            pl.BlockSpec((_EB, HID), lambda i: (i, 0)),
            pl.BlockSpec((_EB, H), lambda i: (i, 0)),
        ],
        out_shape=(_sds((E, HID)), _sds((E, H))),
    )(logits, xs_src, gmax, selt)


def _tc_node(tab, den8, x, wres, bres, w1, b1, w2, b2, selt, km, ln):
    def body(*refs):
        if ln is not None:
            (tab_ref, den_ref, x_ref, wres_ref, bres_ref, w1_ref, b1_ref,
             w2_ref, b2_ref, selt_ref, km_ref, g_ref, be_ref, o_ref) = refs
        else:
            (tab_ref, den_ref, x_ref, wres_ref, bres_ref, w1_ref, b1_ref,
             w2_ref, b2_ref, selt_ref, km_ref, o_ref) = refs
        num = tab_ref[0:N, :] + tab_ref[NP:NP + N, :]
        den = den_ref[0:N, :] + den_ref[N:2 * N, :]
        invb = jnp.dot(1.0 / (den + 1e-16), selt_ref[...],
                       preferred_element_type=_f32)
        agg = jnp.dot(num * invb, km_ref[...], preferred_element_type=_f32)
        out = agg + jnp.dot(x_ref[...], wres_ref[...],
                            preferred_element_type=_f32) + bres_ref[...]
        h1 = jnp.maximum(jnp.dot(out, w1_ref[...],
                                 preferred_element_type=_f32) + b1_ref[...], 0.0)
        out = out + jnp.dot(h1, w2_ref[...],
                            preferred_element_type=_f32) + b2_ref[...]
        if ln is not None:
            mu = jnp.mean(out, axis=1, keepdims=True)
            v = jnp.mean((out - mu) ** 2, axis=1, keepdims=True)
            out = (out - mu) / jnp.sqrt(v + 1e-5) * g_ref[...] + be_ref[...]
        o_ref[...] = out

    args = [tab, den8, x, wres, bres, w1, b1, w2, b2, selt, km]
    if ln is not None:
        args += [ln[0], ln[1]]
    return pl.pallas_call(body, out_shape=_sds((N, OUT)))(*args)


def kernel(mesh_enc, edge_index, edge_attr, params):
    src = edge_index[0]
    dst = edge_index[1]
    zt = jnp.zeros((NP, HID), _f32)

    sel_np = np.kron(np.eye(H, dtype=np.float32),
                     np.ones((OUT, 1), dtype=np.float32))
    sel = jnp.asarray(sel_np)
    selt = jnp.asarray(sel_np.T)
    sel16 = jnp.asarray(np.concatenate(
        [np.eye(H, dtype=np.float32),
         np.zeros((H, OUT - H), dtype=np.float32)], axis=1))
    km = jnp.asarray(np.kron(np.ones((H, 1), dtype=np.float32) / H,
                             np.eye(OUT, dtype=np.float32)))

    wep = params["W_ep"]
    bep = params["b_ep"].reshape(1, HID)

    x = mesh_enc
    e = edge_attr
    for l in range(NL):
        p = params["layers"][l]
        attb = p["att"].reshape(1, H * OUT)
        xs, xd = _tc_project(x, p["Wl"], p["Wr"])
        xs_src, xd_dst = _sc_gather(xs, xd, src, dst)
        enew, logits, gmax = _tc_edge(
            e, xs_src, xd_dst, wep, bep, p["We"], attb, sel,
            p["Weu"], p["beu"].reshape(1, HID), project=(l == 0))
        num_c, ex = _tc_contrib(logits, gmax, xs_src, selt)
        tab = _sc_scatter(num_c, ex.reshape(E * H), dst, zt)
        den8 = jnp.concatenate(
            [tab[N:N + N // 16].reshape(N, H),
             tab[NP + N:NP + N + N // 16].reshape(N, H)], axis=0)
        ln = None
        if l < NL - 1:
            ln = (params["norms"][l]["g"].reshape(1, OUT),
                  params["norms"][l]["b"].reshape(1, OUT))
        x = _tc_node(tab, den8, x, p["Wres"], p["bres"].reshape(1, OUT),
                     p["W1"], p["b1"].reshape(1, FFN),
                     p["W2"], p["b2"].reshape(1, OUT), selt, km, ln)
        e = enew
    return x


# speedup vs baseline: 1.0370x; 1.0280x over previous
import dataclasses
import functools

import jax
import jax.numpy as jnp
import numpy as np
from jax import lax
from jax.experimental import pallas as pl
from jax.experimental.pallas import tpu as pltpu
from jax.experimental.pallas import tpu_sc as plsc

N = 10000
E = 320000
HID = 128
H = 8
OUT = 16
NL = 3
FFN = 128

NC = 2
NS = 16
NW = NC * NS
CH = 128
NCHUNK = E // CH
MAXC = -(-NCHUNK // NW)
NP = N + 640
BZ = 80
NBZ = NP // BZ

_f32 = jnp.float32


def _sds(shape):
    return jax.ShapeDtypeStruct(shape, _f32)


def _sc_params():
    cp = pltpu.CompilerParams()
    if "needs_layout_passes" in pltpu.CompilerParams.__dataclass_fields__:
        cp = dataclasses.replace(cp, needs_layout_passes=False)
    return cp


def _sc_gather(xs, xd, src, dst):
    mesh = plsc.VectorSubcoreMesh(core_axis_name="c", subcore_axis_name="s")

    @functools.partial(
        pl.kernel,
        out_type=(_sds((E, HID)), _sds((E, HID))),
        mesh=mesh,
        scratch_types=[
            pltpu.VMEM((CH,), jnp.int32),
            pltpu.VMEM((CH,), jnp.int32),
            pltpu.VMEM((CH, HID), _f32),
            pltpu.VMEM((CH, HID), _f32),
            pltpu.SemaphoreType.DMA,
            pltpu.SemaphoreType.DMA,
        ],
    )
    def k(xs_hbm, xd_hbm, src_hbm, dst_hbm, os_hbm, od_hbm,
          si_v, di_v, rs_v, rd_v, sem_a, sem_b):
        cid = lax.axis_index("c")
        sid = lax.axis_index("s")
        wid = sid * NC + cid
        per = NCHUNK // NW
        rem = NCHUNK % NW
        start = wid * per + jnp.minimum(wid, rem)
        cnt = per + jnp.where(wid < rem, 1, 0)

        @pl.loop(0, MAXC)
        def _(i):
            @pl.when(i < cnt)
            def _():
                base = (start + i) * CH
                pltpu.sync_copy(src_hbm.at[pl.ds(base, CH)], si_v)
                pltpu.sync_copy(dst_hbm.at[pl.ds(base, CH)], di_v)
                a = pltpu.async_copy(xs_hbm.at[si_v], rs_v, sem_a)
                b = pltpu.async_copy(xd_hbm.at[di_v], rd_v, sem_b)
                a.wait()
                b.wait()
                pltpu.sync_copy(rs_v, os_hbm.at[pl.ds(base, CH)])
                pltpu.sync_copy(rd_v, od_hbm.at[pl.ds(base, CH)])

    return k(xs, xd, src, dst)


def _sc_scatter(num_c, ex_f, dst, zt):
    mesh = plsc.VectorSubcoreMesh(core_axis_name="c", subcore_axis_name="s")

    @functools.partial(
        pl.kernel,
        out_type=_sds((NC * NP, HID)),
        mesh=mesh,
        scratch_types=[
            pltpu.VMEM((CH,), jnp.int32),
            pltpu.VMEM((CH,), jnp.int32),
            pltpu.VMEM((CH, HID), _f32),
            pltpu.VMEM((CH, HID), _f32),
            pltpu.VMEM((CH * H,), _f32),
            pltpu.VMEM_SHARED((NP, HID), _f32),
            pltpu.SemaphoreType.DMA,
            pltpu.SemaphoreType.DMA,
        ],
        compiler_params=_sc_params(),
    )
    def k(nc_hbm, ex_hbm, dst_hbm, zt_hbm, ot_hbm,
          idx_v, didx_v, dat_v, den_v, ex_v, tab_s, sem_a, sem_b):
        cid = lax.axis_index("c")
        sid = lax.axis_index("s")
        wid = sid * NC + cid
        per = NCHUNK // NW
        rem = NCHUNK % NW
        start = wid * per + jnp.minimum(wid, rem)
        cnt = per + jnp.where(wid < rem, 1, 0)
        zv16 = jnp.zeros((16,), _f32)
        lane = lax.iota(jnp.int32, 16)

        @pl.loop(0, CH)
        def _(r):
            @pl.loop(0, HID // 16)
            def _(c):
                den_v[r, pl.ds(c * 16, 16)] = zv16

        @pl.loop(0, -(-NBZ // NS))
        def _(j):
            blk = j * NS + sid

            @pl.when(blk < NBZ)
            def _():
                pltpu.sync_copy(zt_hbm.at[pl.ds(blk * BZ, BZ)],
                                dat_v.at[pl.ds(0, BZ)])
                pltpu.sync_copy(dat_v.at[pl.ds(0, BZ)],
                                tab_s.at[pl.ds(blk * BZ, BZ)])

        plsc.subcore_barrier()

        @pl.loop(0, MAXC)
        def _(i):
            @pl.when(i < cnt)
            def _():
                base = (start + i) * CH
                pltpu.sync_copy(dst_hbm.at[pl.ds(base, CH)], idx_v)
                pltpu.sync_copy(nc_hbm.at[pl.ds(base, CH)], dat_v)
                pltpu.sync_copy(ex_hbm.at[pl.ds(base * H, CH * H)], ex_v)
                add_n = pltpu.async_copy(dat_v, tab_s.at[idx_v], sem_a,
                                         add=True)

                @pl.loop(0, CH // 16)
                def _(g):
                    row = lane + g * 16
                    dvec = idx_v[pl.ds(g * 16, 16)]
                    didx_v[pl.ds(g * 16, 16)] = N + dvec // 16
                    colb = lax.rem(dvec, 16) * H
                    for h in range(H):
                        vals = plsc.load_gather(ex_v, [row * H + h])
                        plsc.store_scatter(den_v, [row, colb + h], vals)

                add_d = pltpu.async_copy(den_v, tab_s.at[didx_v], sem_b,
                                         add=True)
                add_n.wait()
                add_d.wait()

                @pl.loop(0, CH // 16)
                def _(g):
                    row = lane + g * 16
                    dvec = idx_v[pl.ds(g * 16, 16)]
                    colb = lax.rem(dvec, 16) * H
                    for h in range(H):
                        plsc.store_scatter(den_v, [row, colb + h], zv16)

        plsc.subcore_barrier()

        @pl.loop(0, -(-NBZ // NS))
        def _(j):
            blk = j * NS + sid

            @pl.when(blk < NBZ)
            def _():
                pltpu.sync_copy(tab_s.at[pl.ds(blk * BZ, BZ)],
                                dat_v.at[pl.ds(0, BZ)])
                pltpu.sync_copy(dat_v.at[pl.ds(0, BZ)],
                                ot_hbm.at[pl.ds(cid * NP + blk * BZ, BZ)])

    return k(num_c, ex_f, dst, zt)


def _tc_project(x, wl, wr):
    def body(x_ref, wl_ref, wr_ref, xs_ref, xd_ref):
        xv = x_ref[...]
        xs_ref[...] = jnp.dot(xv, wl_ref[...], preferred_element_type=_f32)
        xd_ref[...] = jnp.dot(xv, wr_ref[...], preferred_element_type=_f32)

    return pl.pallas_call(
        body, out_shape=(_sds((N, HID)), _sds((N, HID))),
    )(x, wl, wr)


_EB = 2000


def _tc_edge(e_in, xs_src, xd_dst, wep, bep, we, attb, sel, weu, beu, project):
    nblk = E // _EB
    in_ch = e_in.shape[1]

    def body(e_ref, xs_ref, xd_ref, wep_ref, bep_ref, we_ref, attb_ref,
             sel_ref, weu_ref, beu_ref, enew_ref, log_ref, gmax_ref):
        if project:
            e = jnp.dot(e_ref[...], wep_ref[...],
                        preferred_element_type=_f32) + bep_ref[...]
        else:
            e = e_ref[...]
        ee = jnp.dot(e, we_ref[...], preferred_element_type=_f32)
        m = xs_ref[...] + xd_ref[...] + ee
        g = jnp.where(m >= 0, m, 0.2 * m)
        logits = jnp.dot(g * attb_ref[...], sel_ref[...],
                         preferred_element_type=_f32)
        r = jnp.maximum(g, 0.0)
        enew_ref[...] = e + jnp.dot(r, weu_ref[...],
                                    preferred_element_type=_f32) + beu_ref[...]
        log_ref[...] = logits
        bm = jnp.max(logits, axis=0, keepdims=True)
        i = pl.program_id(0)

        @pl.when(i == 0)
        def _():
            gmax_ref[...] = bm

        @pl.when(i != 0)
        def _():
            gmax_ref[...] = jnp.maximum(gmax_ref[...], bm)

    wspec = lambda s: pl.BlockSpec(s, lambda i: (0, 0))
    return pl.pallas_call(
        body,
        grid=(nblk,),
        in_specs=[
            pl.BlockSpec((_EB, in_ch), lambda i: (i, 0)),
            pl.BlockSpec((_EB, HID), lambda i: (i, 0)),
            pl.BlockSpec((_EB, HID), lambda i: (i, 0)),
            wspec((16, HID)), wspec((1, HID)), wspec((HID, HID)),
            wspec((1, HID)), wspec((HID, H)), wspec((HID, HID)),
            wspec((1, HID)),
        ],
        out_specs=[
            pl.BlockSpec((_EB, HID), lambda i: (i, 0)),
            pl.BlockSpec((_EB, H), lambda i: (i, 0)),
            pl.BlockSpec((1, H), lambda i: (0, 0)),
        ],
        out_shape=(_sds((E, HID)), _sds((E, H)), _sds((1, H))),
    )(e_in, xs_src, xd_dst, wep, bep, we, attb, sel, weu, beu)


def _tc_contrib(logits, gmax, xs_src, selt):
    nblk = E // _EB

    def body(log_ref, xs_ref, gmax_ref, selt_ref, num_ref, ex_ref):
        ex = jnp.exp(log_ref[...] - gmax_ref[...])
        exb = jnp.dot(ex, selt_ref[...], preferred_element_type=_f32)
        num_ref[...] = exb * xs_ref[...]
        ex_ref[...] = ex

    return pl.pallas_call(
        body,
        grid=(nblk,),
        in_specs=[
            pl.BlockSpec((_EB, H), lambda i: (i, 0)),
            pl.BlockSpec((_EB, HID), lambda i: (i, 0)),
            pl.BlockSpec((1, H), lambda i: (0, 0)),
            pl.BlockSpec((H, HID), lambda i: (0, 0)),
        ],
        out_specs=[
            pl.BlockSpec((_EB, HID), lambda i: (i, 0)),
            pl.BlockSpec((_EB, H), lambda i: (i, 0)),
        ],
        out_shape=(_sds((E, HID)), _sds((E, H))),
    )(logits, xs_src, gmax, selt)


def _tc_node(tab, den8, x, wres, bres, w1, b1, w2, b2, selt, km, ln):
    def body(*refs):
        if ln is not None:
            (tab_ref, den_ref, x_ref, wres_ref, bres_ref, w1_ref, b1_ref,
             w2_ref, b2_ref, selt_ref, km_ref, g_ref, be_ref, o_ref) = refs
        else:
            (tab_ref, den_ref, x_ref, wres_ref, bres_ref, w1_ref, b1_ref,
             w2_ref, b2_ref, selt_ref, km_ref, o_ref) = refs
        num = tab_ref[0:N, :] + tab_ref[NP:NP + N, :]
        den = den_ref[0:N, :] + den_ref[N:2 * N, :]
        invb = jnp.dot(1.0 / (den + 1e-16), selt_ref[...],
                       preferred_element_type=_f32)
        agg = jnp.dot(num * invb, km_ref[...], preferred_element_type=_f32)
        out = agg + jnp.dot(x_ref[...], wres_ref[...],
                            preferred_element_type=_f32) + bres_ref[...]
        h1 = jnp.maximum(jnp.dot(out, w1_ref[...],
                                 preferred_element_type=_f32) + b1_ref[...], 0.0)
        out = out + jnp.dot(h1, w2_ref[...],
                            preferred_element_type=_f32) + b2_ref[...]
        if ln is not None:
            mu = jnp.mean(out, axis=1, keepdims=True)
            v = jnp.mean((out - mu) ** 2, axis=1, keepdims=True)
            out = (out - mu) / jnp.sqrt(v + 1e-5) * g_ref[...] + be_ref[...]
        o_ref[...] = out

    args = [tab, den8, x, wres, bres, w1, b1, w2, b2, selt, km]
    if ln is not None:
        args += [ln[0], ln[1]]
    return pl.pallas_call(body, out_shape=_sds((N, OUT)))(*args)


def kernel(mesh_enc, edge_index, edge_attr, params):
    src = edge_index[0]
    dst = edge_index[1]
    zt = jnp.zeros((NP, HID), _f32)

    sel_np = np.kron(np.eye(H, dtype=np.float32),
                     np.ones((OUT, 1), dtype=np.float32))
    sel = jnp.asarray(sel_np)
    selt = jnp.asarray(sel_np.T)
    sel16 = jnp.asarray(np.concatenate(
        [np.eye(H, dtype=np.float32),
         np.zeros((H, OUT - H), dtype=np.float32)], axis=1))
    km = jnp.asarray(np.kron(np.ones((H, 1), dtype=np.float32) / H,
                             np.eye(OUT, dtype=np.float32)))

    wep = params["W_ep"]
    bep = params["b_ep"].reshape(1, HID)

    x = mesh_enc
    e = edge_attr
    for l in range(NL):
        p = params["layers"][l]
        attb = p["att"].reshape(1, H * OUT)
        xs, xd = _tc_project(x, p["Wl"], p["Wr"])
        xs_src, xd_dst = _sc_gather(xs, xd, src, dst)
        enew, logits, gmax = _tc_edge(
            e, xs_src, xd_dst, wep, bep, p["We"], attb, sel,
            p["Weu"], p["beu"].reshape(1, HID), project=(l == 0))
        num_c, ex = _tc_contrib(logits, gmax, xs_src, selt)
        tab = _sc_scatter(num_c, ex.reshape(E * H), dst, zt)
        den8 = jnp.concatenate(
            [tab[N:N + N // 16].reshape(N, H),
             tab[NP + N:NP + N + N // 16].reshape(N, H)], axis=0)
        ln = None
        if l < NL - 1:
            ln = (params["norms"][l]["g"].reshape(1, OUT),
                  params["norms"][l]["b"].reshape(1, OUT))
        x = _tc_node(tab, den8, x, p["Wres"], p["bres"].reshape(1, OUT),
                     p["W1"], p["b1"].reshape(1, FFN),
                     p["W2"], p["b2"].reshape(1, OUT), selt, km, ln)
        e = enew
    return x


# speedup vs baseline: 1.1360x; 1.0955x over previous
import dataclasses
import functools

import jax
import jax.numpy as jnp
import numpy as np
from jax import lax
from jax.experimental import pallas as pl
from jax.experimental.pallas import tpu as pltpu
from jax.experimental.pallas import tpu_sc as plsc

N = 10000
E = 320000
HID = 128
H = 8
OUT = 16
NL = 3
FFN = 128

NC = 2
NS = 16
NW = NC * NS
CH = 128
NCHUNK = E // CH
MAXC = -(-NCHUNK // NW)
NP = N + 640
BZ = 80
NBZ = NP // BZ

_f32 = jnp.float32


def _sds(shape):
    return jax.ShapeDtypeStruct(shape, _f32)


def _sc_params():
    cp = pltpu.CompilerParams()
    if "needs_layout_passes" in pltpu.CompilerParams.__dataclass_fields__:
        cp = dataclasses.replace(cp, needs_layout_passes=False)
    return cp


def _sc_gather(xs, xd, src, dst):
    mesh = plsc.VectorSubcoreMesh(core_axis_name="c", subcore_axis_name="s")

    @functools.partial(
        pl.kernel,
        out_type=(_sds((E, HID)), _sds((E, HID))),
        mesh=mesh,
        scratch_types=[
            pltpu.VMEM((CH,), jnp.int32),
            pltpu.VMEM((CH,), jnp.int32),
            pltpu.VMEM((CH, HID), _f32),
            pltpu.VMEM((CH, HID), _f32),
            pltpu.SemaphoreType.DMA,
            pltpu.SemaphoreType.DMA,
        ],
    )
    def k(xs_hbm, xd_hbm, src_hbm, dst_hbm, os_hbm, od_hbm,
          si_v, di_v, rs_v, rd_v, sem_a, sem_b):
        cid = lax.axis_index("c")
        sid = lax.axis_index("s")
        wid = sid * NC + cid
        per = NCHUNK // NW
        rem = NCHUNK % NW
        start = wid * per + jnp.minimum(wid, rem)
        cnt = per + jnp.where(wid < rem, 1, 0)

        @pl.loop(0, MAXC)
        def _(i):
            @pl.when(i < cnt)
            def _():
                base = (start + i) * CH
                ia = pltpu.async_copy(src_hbm.at[pl.ds(base, CH)], si_v, sem_a)
                ib = pltpu.async_copy(dst_hbm.at[pl.ds(base, CH)], di_v, sem_b)
                ia.wait()
                ib.wait()
                a = pltpu.async_copy(xs_hbm.at[si_v], rs_v, sem_a)
                b = pltpu.async_copy(xd_hbm.at[di_v], rd_v, sem_b)
                a.wait()
                b.wait()
                sa = pltpu.async_copy(rs_v, os_hbm.at[pl.ds(base, CH)], sem_a)
                sb = pltpu.async_copy(rd_v, od_hbm.at[pl.ds(base, CH)], sem_b)
                sa.wait()
                sb.wait()

    return k(xs, xd, src, dst)


def _sc_scatter(num_c, ex_f, dst, zt):
    mesh = plsc.VectorSubcoreMesh(core_axis_name="c", subcore_axis_name="s")

    @functools.partial(
        pl.kernel,
        out_type=_sds((NC * NP, HID)),
        mesh=mesh,
        scratch_types=[
            pltpu.VMEM((CH,), jnp.int32),
            pltpu.VMEM((CH,), jnp.int32),
            pltpu.VMEM((CH, HID), _f32),
            pltpu.VMEM((CH, HID), _f32),
            pltpu.VMEM((CH * H,), _f32),
            pltpu.VMEM_SHARED((NP, HID), _f32),
            pltpu.SemaphoreType.DMA,
            pltpu.SemaphoreType.DMA,
        ],
        compiler_params=_sc_params(),
    )
    def k(nc_hbm, ex_hbm, dst_hbm, zt_hbm, ot_hbm,
          idx_v, didx_v, dat_v, den_v, ex_v, tab_s, sem_a, sem_b):
        cid = lax.axis_index("c")
        sid = lax.axis_index("s")
        wid = sid * NC + cid
        per = NCHUNK // NW
        rem = NCHUNK % NW
        start = wid * per + jnp.minimum(wid, rem)
        cnt = per + jnp.where(wid < rem, 1, 0)
        zv16 = jnp.zeros((16,), _f32)
        lane = lax.iota(jnp.int32, 16)

        @pl.loop(0, CH)
        def _(r):
            @pl.loop(0, HID // 16)
            def _(c):
                den_v[r, pl.ds(c * 16, 16)] = zv16

        @pl.loop(0, -(-NBZ // NS))
        def _(j):
            blk = j * NS + sid

            @pl.when(blk < NBZ)
            def _():
                pltpu.sync_copy(zt_hbm.at[pl.ds(blk * BZ, BZ)],
                                dat_v.at[pl.ds(0, BZ)])
                pltpu.sync_copy(dat_v.at[pl.ds(0, BZ)],
                                tab_s.at[pl.ds(blk * BZ, BZ)])

        plsc.subcore_barrier()

        @pl.loop(0, MAXC)
        def _(i):
            @pl.when(i < cnt)
            def _():
                base = (start + i) * CH
                li = pltpu.async_copy(dst_hbm.at[pl.ds(base, CH)], idx_v,
                                      sem_a)
                ld = pltpu.async_copy(nc_hbm.at[pl.ds(base, CH)], dat_v,
                                      sem_b)
                le = pltpu.async_copy(ex_hbm.at[pl.ds(base * H, CH * H)],
                                      ex_v, sem_a)
                li.wait()
                ld.wait()
                le.wait()
                add_n = pltpu.async_copy(dat_v, tab_s.at[idx_v], sem_a,
                                         add=True)

                @pl.loop(0, CH // 16)
                def _(g):
                    row = lane + g * 16
                    dvec = idx_v[pl.ds(g * 16, 16)]
                    didx_v[pl.ds(g * 16, 16)] = N + dvec // 16
                    colb = lax.rem(dvec, 16) * H
                    for h in range(H):
                        vals = plsc.load_gather(ex_v, [row * H + h])
                        plsc.store_scatter(den_v, [row, colb + h], vals)

                add_d = pltpu.async_copy(den_v, tab_s.at[didx_v], sem_b,
                                         add=True)
                add_n.wait()
                add_d.wait()

                @pl.loop(0, CH // 16)
                def _(g):
                    row = lane + g * 16
                    dvec = idx_v[pl.ds(g * 16, 16)]
                    colb = lax.rem(dvec, 16) * H
                    for h in range(H):
                        plsc.store_scatter(den_v, [row, colb + h], zv16)

        plsc.subcore_barrier()

        @pl.loop(0, -(-NBZ // NS))
        def _(j):
            blk = j * NS + sid

            @pl.when(blk < NBZ)
            def _():
                pltpu.sync_copy(tab_s.at[pl.ds(blk * BZ, BZ)],
                                dat_v.at[pl.ds(0, BZ)])
                pltpu.sync_copy(dat_v.at[pl.ds(0, BZ)],
                                ot_hbm.at[pl.ds(cid * NP + blk * BZ, BZ)])

    return k(num_c, ex_f, dst, zt)


def _tc_project(x, wl, wr):
    def body(x_ref, wl_ref, wr_ref, xs_ref, xd_ref):
        xv = x_ref[...]
        xs_ref[...] = jnp.dot(xv, wl_ref[...], preferred_element_type=_f32)
        xd_ref[...] = jnp.dot(xv, wr_ref[...], preferred_element_type=_f32)

    return pl.pallas_call(
        body, out_shape=(_sds((N, HID)), _sds((N, HID))),
    )(x, wl, wr)


_EB = 2000


def _tc_edge(e_in, xs_src, xd_dst, wep, bep, we, attb, sel, weu, beu, project):
    nblk = E // _EB
    in_ch = e_in.shape[1]

    def body(e_ref, xs_ref, xd_ref, wep_ref, bep_ref, we_ref, attb_ref,
             sel_ref, weu_ref, beu_ref, enew_ref, log_ref, gmax_ref):
        if project:
            e = jnp.dot(e_ref[...], wep_ref[...],
                        preferred_element_type=_f32) + bep_ref[...]
        else:
            e = e_ref[...]
        ee = jnp.dot(e, we_ref[...], preferred_element_type=_f32)
        m = xs_ref[...] + xd_ref[...] + ee
        g = jnp.where(m >= 0, m, 0.2 * m)
        logits = jnp.dot(g * attb_ref[...], sel_ref[...],
                         preferred_element_type=_f32)
        r = jnp.maximum(g, 0.0)
        enew_ref[...] = e + jnp.dot(r, weu_ref[...],
                                    preferred_element_type=_f32) + beu_ref[...]
        log_ref[...] = logits
        bm = jnp.max(logits, axis=0, keepdims=True)
        i = pl.program_id(0)

        @pl.when(i == 0)
        def _():
            gmax_ref[...] = bm

        @pl.when(i != 0)
        def _():
            gmax_ref[...] = jnp.maximum(gmax_ref[...], bm)

    wspec = lambda s: pl.BlockSpec(s, lambda i: (0, 0))
    return pl.pallas_call(
        body,
        grid=(nblk,),
        in_specs=[
            pl.BlockSpec((_EB, in_ch), lambda i: (i, 0)),
            pl.BlockSpec((_EB, HID), lambda i: (i, 0)),
            pl.BlockSpec((_EB, HID), lambda i: (i, 0)),
            wspec((16, HID)), wspec((1, HID)), wspec((HID, HID)),
            wspec((1, HID)), wspec((HID, H)), wspec((HID, HID)),
            wspec((1, HID)),
        ],
        out_specs=[
            pl.BlockSpec((_EB, HID), lambda i: (i, 0)),
            pl.BlockSpec((_EB, H), lambda i: (i, 0)),
            pl.BlockSpec((1, H), lambda i: (0, 0)),
        ],
        out_shape=(_sds((E, HID)), _sds((E, H)), _sds((1, H))),
    )(e_in, xs_src, xd_dst, wep, bep, we, attb, sel, weu, beu)


def _tc_contrib(logits, gmax, xs_src, selt):
    nblk = E // _EB

    def body(log_ref, xs_ref, gmax_ref, selt_ref, num_ref, ex_ref):
        ex = jnp.exp(log_ref[...] - gmax_ref[...])
        exb = jnp.dot(ex, selt_ref[...], preferred_element_type=_f32)
        num_ref[...] = exb * xs_ref[...]
        ex_ref[...] = ex

    return pl.pallas_call(
        body,
        grid=(nblk,),
        in_specs=[
            pl.BlockSpec((_EB, H), lambda i: (i, 0)),
            pl.BlockSpec((_EB, HID), lambda i: (i, 0)),
            pl.BlockSpec((1, H), lambda i: (0, 0)),
            pl.BlockSpec((H, HID), lambda i: (0, 0)),
        ],
        out_specs=[
            pl.BlockSpec((_EB, HID), lambda i: (i, 0)),
            pl.BlockSpec((_EB, H), lambda i: (i, 0)),
        ],
        out_shape=(_sds((E, HID)), _sds((E, H))),
    )(logits, xs_src, gmax, selt)


def _tc_node(tab, den8, x, wres, bres, w1, b1, w2, b2, selt, km, ln):
    def body(*refs):
        if ln is not None:
            (tab_ref, den_ref, x_ref, wres_ref, bres_ref, w1_ref, b1_ref,
             w2_ref, b2_ref, selt_ref, km_ref, g_ref, be_ref, o_ref) = refs
        else:
            (tab_ref, den_ref, x_ref, wres_ref, bres_ref, w1_ref, b1_ref,
             w2_ref, b2_ref, selt_ref, km_ref, o_ref) = refs
        num = tab_ref[0:N, :] + tab_ref[NP:NP + N, :]
        den = den_ref[0:N, :] + den_ref[N:2 * N, :]
        invb = jnp.dot(1.0 / (den + 1e-16), selt_ref[...],
                       preferred_element_type=_f32)
        agg = jnp.dot(num * invb, km_ref[...], preferred_element_type=_f32)
        out = agg + jnp.dot(x_ref[...], wres_ref[...],
                            preferred_element_type=_f32) + bres_ref[...]
        h1 = jnp.maximum(jnp.dot(out, w1_ref[...],
                                 preferred_element_type=_f32) + b1_ref[...], 0.0)
        out = out + jnp.dot(h1, w2_ref[...],
                            preferred_element_type=_f32) + b2_ref[...]
        if ln is not None:
            mu = jnp.mean(out, axis=1, keepdims=True)
            v = jnp.mean((out - mu) ** 2, axis=1, keepdims=True)
            out = (out - mu) / jnp.sqrt(v + 1e-5) * g_ref[...] + be_ref[...]
        o_ref[...] = out

    args = [tab, den8, x, wres, bres, w1, b1, w2, b2, selt, km]
    if ln is not None:
        args += [ln[0], ln[1]]
    return pl.pallas_call(body, out_shape=_sds((N, OUT)))(*args)


def kernel(mesh_enc, edge_index, edge_attr, params):
    src = edge_index[0]
    dst = edge_index[1]
    zt = jnp.zeros((NP, HID), _f32)

    sel_np = np.kron(np.eye(H, dtype=np.float32),
                     np.ones((OUT, 1), dtype=np.float32))
    sel = jnp.asarray(sel_np)
    selt = jnp.asarray(sel_np.T)
    sel16 = jnp.asarray(np.concatenate(
        [np.eye(H, dtype=np.float32),
         np.zeros((H, OUT - H), dtype=np.float32)], axis=1))
    km = jnp.asarray(np.kron(np.ones((H, 1), dtype=np.float32) / H,
                             np.eye(OUT, dtype=np.float32)))

    wep = params["W_ep"]
    bep = params["b_ep"].reshape(1, HID)

    x = mesh_enc
    e = edge_attr
    for l in range(NL):
        p = params["layers"][l]
        attb = p["att"].reshape(1, H * OUT)
        xs, xd = _tc_project(x, p["Wl"], p["Wr"])
        xs_src, xd_dst = _sc_gather(xs, xd, src, dst)
        enew, logits, gmax = _tc_edge(
            e, xs_src, xd_dst, wep, bep, p["We"], attb, sel,
            p["Weu"], p["beu"].reshape(1, HID), project=(l == 0))
        num_c, ex = _tc_contrib(logits, gmax, xs_src, selt)
        tab = _sc_scatter(num_c, ex.reshape(E * H), dst, zt)
        den8 = jnp.concatenate(
            [tab[N:N + N // 16].reshape(N, H),
             tab[NP + N:NP + N + N // 16].reshape(N, H)], axis=0)
        ln = None
        if l < NL - 1:
            ln = (params["norms"][l]["g"].reshape(1, OUT),
                  params["norms"][l]["b"].reshape(1, OUT))
        x = _tc_node(tab, den8, x, p["Wres"], p["bres"].reshape(1, OUT),
                     p["W1"], p["b1"].reshape(1, FFN),
                     p["W2"], p["b2"].reshape(1, OUT), selt, km, ln)
        e = enew
    return x


# speedup vs baseline: 1.1873x; 1.0452x over previous
import dataclasses
import functools

import jax
import jax.numpy as jnp
import numpy as np
from jax import lax
from jax.experimental import pallas as pl
from jax.experimental.pallas import tpu as pltpu
from jax.experimental.pallas import tpu_sc as plsc

N = 10000
E = 320000
HID = 128
H = 8
OUT = 16
NL = 3
FFN = 128

NC = 2
NS = 16
NW = NC * NS
CH = 128
NCHUNK = E // CH
MAXC = -(-NCHUNK // NW)
NP = N + 640
BZ = 80
NBZ = NP // BZ

_f32 = jnp.float32


def _sds(shape):
    return jax.ShapeDtypeStruct(shape, _f32)


def _sc_params():
    cp = pltpu.CompilerParams()
    if "needs_layout_passes" in pltpu.CompilerParams.__dataclass_fields__:
        cp = dataclasses.replace(cp, needs_layout_passes=False)
    return cp


def _sc_gather(xs, xd, src, dst):
    mesh = plsc.VectorSubcoreMesh(core_axis_name="c", subcore_axis_name="s")

    @functools.partial(
        pl.kernel,
        out_type=(_sds((E, HID)), _sds((E, HID))),
        mesh=mesh,
        scratch_types=[
            pltpu.VMEM((CH,), jnp.int32),
            pltpu.VMEM((CH,), jnp.int32),
            pltpu.VMEM((CH, HID), _f32),
            pltpu.VMEM((CH, HID), _f32),
            pltpu.VMEM((CH, HID), _f32),
            pltpu.VMEM((CH, HID), _f32),
            pltpu.SemaphoreType.DMA,
            pltpu.SemaphoreType.DMA,
            pltpu.SemaphoreType.DMA,
            pltpu.SemaphoreType.DMA,
            pltpu.SemaphoreType.DMA,
            pltpu.SemaphoreType.DMA,
        ],
    )
    def k(xs_hbm, xd_hbm, src_hbm, dst_hbm, os_hbm, od_hbm,
          si_v, di_v, rs0, rd0, rs1, rd1,
          sem_a, sem_b, ss0a, ss0b, ss1a, ss1b):
        cid = lax.axis_index("c")
        sid = lax.axis_index("s")
        wid = sid * NC + cid
        per = NCHUNK // NW
        rem = NCHUNK % NW
        start = wid * per + jnp.minimum(wid, rem)
        cnt = per + jnp.where(wid < rem, 1, 0)

        def drain_stores(rs, rd, ssa, ssb):
            pltpu.make_async_copy(rs, os_hbm.at[pl.ds(0, CH)], ssa).wait()
            pltpu.make_async_copy(rd, od_hbm.at[pl.ds(0, CH)], ssb).wait()

        def do_chunk(i, rs, rd, ssa, ssb):
            base = (start + i) * CH
            ia = pltpu.async_copy(src_hbm.at[pl.ds(base, CH)], si_v, sem_a)
            ib = pltpu.async_copy(dst_hbm.at[pl.ds(base, CH)], di_v, sem_b)
            ia.wait()
            ib.wait()
            a = pltpu.async_copy(xs_hbm.at[si_v], rs, sem_a)
            b = pltpu.async_copy(xd_hbm.at[di_v], rd, sem_b)
            a.wait()
            b.wait()
            pltpu.async_copy(rs, os_hbm.at[pl.ds(base, CH)], ssa)
            pltpu.async_copy(rd, od_hbm.at[pl.ds(base, CH)], ssb)

        @pl.loop(0, MAXC)
        def _(i):
            @pl.when((i < cnt) & (i % 2 == 0))
            def _():
                @pl.when(i >= 2)
                def _():
                    drain_stores(rs0, rd0, ss0a, ss0b)

                do_chunk(i, rs0, rd0, ss0a, ss0b)

            @pl.when((i < cnt) & (i % 2 == 1))
            def _():
                @pl.when(i >= 2)
                def _():
                    drain_stores(rs1, rd1, ss1a, ss1b)

                do_chunk(i, rs1, rd1, ss1a, ss1b)

        @pl.when((cnt >= 1) & ((cnt - 1) % 2 == 0))
        def _():
            drain_stores(rs0, rd0, ss0a, ss0b)

        @pl.when((cnt >= 1) & ((cnt - 1) % 2 == 1))
        def _():
            drain_stores(rs1, rd1, ss1a, ss1b)

        @pl.when((cnt >= 2) & ((cnt - 2) % 2 == 0))
        def _():
            drain_stores(rs0, rd0, ss0a, ss0b)

        @pl.when((cnt >= 2) & ((cnt - 2) % 2 == 1))
        def _():
            drain_stores(rs1, rd1, ss1a, ss1b)

    return k(xs, xd, src, dst)


def _sc_scatter(num_c, ex_f, dst, zt):
    mesh = plsc.VectorSubcoreMesh(core_axis_name="c", subcore_axis_name="s")

    @functools.partial(
        pl.kernel,
        out_type=_sds((NC * NP, HID)),
        mesh=mesh,
        scratch_types=[
            pltpu.VMEM((CH,), jnp.int32),
            pltpu.VMEM((CH,), jnp.int32),
            pltpu.VMEM((CH, HID), _f32),
            pltpu.VMEM((CH, HID), _f32),
            pltpu.VMEM((CH * H,), _f32),
            pltpu.VMEM_SHARED((NP, HID), _f32),
            pltpu.SemaphoreType.DMA,
            pltpu.SemaphoreType.DMA,
        ],
        compiler_params=_sc_params(),
    )
    def k(nc_hbm, ex_hbm, dst_hbm, zt_hbm, ot_hbm,
          idx_v, didx_v, dat_v, den_v, ex_v, tab_s, sem_a, sem_b):
        cid = lax.axis_index("c")
        sid = lax.axis_index("s")
        wid = sid * NC + cid
        per = NCHUNK // NW
        rem = NCHUNK % NW
        start = wid * per + jnp.minimum(wid, rem)
        cnt = per + jnp.where(wid < rem, 1, 0)
        zv16 = jnp.zeros((16,), _f32)
        lane = lax.iota(jnp.int32, 16)

        @pl.loop(0, CH)
        def _(r):
            @pl.loop(0, HID // 16)
            def _(c):
                den_v[r, pl.ds(c * 16, 16)] = zv16

        @pl.loop(0, -(-NBZ // NS))
        def _(j):
            blk = j * NS + sid

            @pl.when(blk < NBZ)
            def _():
                pltpu.sync_copy(zt_hbm.at[pl.ds(blk * BZ, BZ)],
                                dat_v.at[pl.ds(0, BZ)])
                pltpu.sync_copy(dat_v.at[pl.ds(0, BZ)],
                                tab_s.at[pl.ds(blk * BZ, BZ)])

        plsc.subcore_barrier()

        @pl.loop(0, MAXC)
        def _(i):
            @pl.when(i < cnt)
            def _():
                base = (start + i) * CH
                li = pltpu.async_copy(dst_hbm.at[pl.ds(base, CH)], idx_v,
                                      sem_a)
                ld = pltpu.async_copy(nc_hbm.at[pl.ds(base, CH)], dat_v,
                                      sem_b)
                le = pltpu.async_copy(ex_hbm.at[pl.ds(base * H, CH * H)],
                                      ex_v, sem_a)
                li.wait()
                ld.wait()
                le.wait()
                add_n = pltpu.async_copy(dat_v, tab_s.at[idx_v], sem_a,
                                         add=True)

                @pl.loop(0, CH // 16)
                def _(g):
                    row = lane + g * 16
                    dvec = idx_v[pl.ds(g * 16, 16)]
                    didx_v[pl.ds(g * 16, 16)] = N + dvec // 16
                    colb = lax.rem(dvec, 16) * H
                    for h in range(H):
                        vals = plsc.load_gather(ex_v, [row * H + h])
                        plsc.store_scatter(den_v, [row, colb + h], vals)

                add_d = pltpu.async_copy(den_v, tab_s.at[didx_v], sem_b,
                                         add=True)
                add_n.wait()
                add_d.wait()

                @pl.loop(0, CH // 16)
                def _(g):
                    row = lane + g * 16
                    dvec = idx_v[pl.ds(g * 16, 16)]
                    colb = lax.rem(dvec, 16) * H
                    for h in range(H):
                        plsc.store_scatter(den_v, [row, colb + h], zv16)

        plsc.subcore_barrier()

        @pl.loop(0, -(-NBZ // NS))
        def _(j):
            blk = j * NS + sid

            @pl.when(blk < NBZ)
            def _():
                pltpu.sync_copy(tab_s.at[pl.ds(blk * BZ, BZ)],
                                dat_v.at[pl.ds(0, BZ)])
                pltpu.sync_copy(dat_v.at[pl.ds(0, BZ)],
                                ot_hbm.at[pl.ds(cid * NP + blk * BZ, BZ)])

    return k(num_c, ex_f, dst, zt)


def _tc_project(x, wl, wr):
    def body(x_ref, wl_ref, wr_ref, xs_ref, xd_ref):
        xv = x_ref[...]
        xs_ref[...] = jnp.dot(xv, wl_ref[...], preferred_element_type=_f32)
        xd_ref[...] = jnp.dot(xv, wr_ref[...], preferred_element_type=_f32)

    return pl.pallas_call(
        body, out_shape=(_sds((N, HID)), _sds((N, HID))),
    )(x, wl, wr)


_EB = 2000


def _tc_edge(e_in, xs_src, xd_dst, wep, bep, we, attb, sel, weu, beu, project):
    nblk = E // _EB
    in_ch = e_in.shape[1]

    def body(e_ref, xs_ref, xd_ref, wep_ref, bep_ref, we_ref, attb_ref,
             sel_ref, weu_ref, beu_ref, enew_ref, log_ref, gmax_ref):
        if project:
            e = jnp.dot(e_ref[...], wep_ref[...],
                        preferred_element_type=_f32) + bep_ref[...]
        else:
            e = e_ref[...]
        ee = jnp.dot(e, we_ref[...], preferred_element_type=_f32)
        m = xs_ref[...] + xd_ref[...] + ee
        g = jnp.where(m >= 0, m, 0.2 * m)
        logits = jnp.dot(g * attb_ref[...], sel_ref[...],
                         preferred_element_type=_f32)
        r = jnp.maximum(g, 0.0)
        enew_ref[...] = e + jnp.dot(r, weu_ref[...],
                                    preferred_element_type=_f32) + beu_ref[...]
        log_ref[...] = logits
        bm = jnp.max(logits, axis=0, keepdims=True)
        i = pl.program_id(0)

        @pl.when(i == 0)
        def _():
            gmax_ref[...] = bm

        @pl.when(i != 0)
        def _():
            gmax_ref[...] = jnp.maximum(gmax_ref[...], bm)

    wspec = lambda s: pl.BlockSpec(s, lambda i: (0, 0))
    return pl.pallas_call(
        body,
        grid=(nblk,),
        in_specs=[
            pl.BlockSpec((_EB, in_ch), lambda i: (i, 0)),
            pl.BlockSpec((_EB, HID), lambda i: (i, 0)),
            pl.BlockSpec((_EB, HID), lambda i: (i, 0)),
            wspec((16, HID)), wspec((1, HID)), wspec((HID, HID)),
            wspec((1, HID)), wspec((HID, H)), wspec((HID, HID)),
            wspec((1, HID)),
        ],
        out_specs=[
            pl.BlockSpec((_EB, HID), lambda i: (i, 0)),
            pl.BlockSpec((_EB, H), lambda i: (i, 0)),
            pl.BlockSpec((1, H), lambda i: (0, 0)),
        ],
        out_shape=(_sds((E, HID)), _sds((E, H)), _sds((1, H))),
    )(e_in, xs_src, xd_dst, wep, bep, we, attb, sel, weu, beu)


def _tc_contrib(logits, gmax, xs_src, selt):
    nblk = E // _EB

    def body(log_ref, xs_ref, gmax_ref, selt_ref, num_ref, ex_ref):
        ex = jnp.exp(log_ref[...] - gmax_ref[...])
        exb = jnp.dot(ex, selt_ref[...], preferred_element_type=_f32)
        num_ref[...] = exb * xs_ref[...]
        ex_ref[...] = ex

    return pl.pallas_call(
        body,
        grid=(nblk,),
        in_specs=[
            pl.BlockSpec((_EB, H), lambda i: (i, 0)),
            pl.BlockSpec((_EB, HID), lambda i: (i, 0)),
            pl.BlockSpec((1, H), lambda i: (0, 0)),
            pl.BlockSpec((H, HID), lambda i: (0, 0)),
        ],
        out_specs=[
            pl.BlockSpec((_EB, HID), lambda i: (i, 0)),
            pl.BlockSpec((_EB, H), lambda i: (i, 0)),
        ],
        out_shape=(_sds((E, HID)), _sds((E, H))),
    )(logits, xs_src, gmax, selt)


def _tc_node(tab, den8, x, wres, bres, w1, b1, w2, b2, selt, km, ln):
    def body(*refs):
        if ln is not None:
            (tab_ref, den_ref, x_ref, wres_ref, bres_ref, w1_ref, b1_ref,
             w2_ref, b2_ref, selt_ref, km_ref, g_ref, be_ref, o_ref) = refs
        else:
            (tab_ref, den_ref, x_ref, wres_ref, bres_ref, w1_ref, b1_ref,
             w2_ref, b2_ref, selt_ref, km_ref, o_ref) = refs
        num = tab_ref[0:N, :] + tab_ref[NP:NP + N, :]
        den = den_ref[0:N, :] + den_ref[N:2 * N, :]
        invb = jnp.dot(1.0 / (den + 1e-16), selt_ref[...],
                       preferred_element_type=_f32)
        agg = jnp.dot(num * invb, km_ref[...], preferred_element_type=_f32)
        out = agg + jnp.dot(x_ref[...], wres_ref[...],
                            preferred_element_type=_f32) + bres_ref[...]
        h1 = jnp.maximum(jnp.dot(out, w1_ref[...],
                                 preferred_element_type=_f32) + b1_ref[...], 0.0)
        out = out + jnp.dot(h1, w2_ref[...],
                            preferred_element_type=_f32) + b2_ref[...]
        if ln is not None:
            mu = jnp.mean(out, axis=1, keepdims=True)
            v = jnp.mean((out - mu) ** 2, axis=1, keepdims=True)
            out = (out - mu) / jnp.sqrt(v + 1e-5) * g_ref[...] + be_ref[...]
        o_ref[...] = out

    args = [tab, den8, x, wres, bres, w1, b1, w2, b2, selt, km]
    if ln is not None:
        args += [ln[0], ln[1]]
    return pl.pallas_call(body, out_shape=_sds((N, OUT)))(*args)


def kernel(mesh_enc, edge_index, edge_attr, params):
    src = edge_index[0]
    dst = edge_index[1]
    zt = jnp.zeros((NP, HID), _f32)

    sel_np = np.kron(np.eye(H, dtype=np.float32),
                     np.ones((OUT, 1), dtype=np.float32))
    sel = jnp.asarray(sel_np)
    selt = jnp.asarray(sel_np.T)
    sel16 = jnp.asarray(np.concatenate(
        [np.eye(H, dtype=np.float32),
         np.zeros((H, OUT - H), dtype=np.float32)], axis=1))
    km = jnp.asarray(np.kron(np.ones((H, 1), dtype=np.float32) / H,
                             np.eye(OUT, dtype=np.float32)))

    wep = params["W_ep"]
    bep = params["b_ep"].reshape(1, HID)

    x = mesh_enc
    e = edge_attr
    for l in range(NL):
        p = params["layers"][l]
        attb = p["att"].reshape(1, H * OUT)
        xs, xd = _tc_project(x, p["Wl"], p["Wr"])
        xs_src, xd_dst = _sc_gather(xs, xd, src, dst)
        enew, logits, gmax = _tc_edge(
            e, xs_src, xd_dst, wep, bep, p["We"], attb, sel,
            p["Weu"], p["beu"].reshape(1, HID), project=(l == 0))
        num_c, ex = _tc_contrib(logits, gmax, xs_src, selt)
        tab = _sc_scatter(num_c, ex.reshape(E * H), dst, zt)
        den8 = jnp.concatenate(
            [tab[N:N + N // 16].reshape(N, H),
             tab[NP + N:NP + N + N // 16].reshape(N, H)], axis=0)
        ln = None
        if l < NL - 1:
            ln = (params["norms"][l]["g"].reshape(1, OUT),
                  params["norms"][l]["b"].reshape(1, OUT))
        x = _tc_node(tab, den8, x, p["Wres"], p["bres"].reshape(1, OUT),
                     p["W1"], p["b1"].reshape(1, FFN),
                     p["W2"], p["b2"].reshape(1, OUT), selt, km, ln)
        e = enew
    return x
